# baseline (speedup 1.0000x reference)
import functools
import math

import jax
import jax.numpy as jnp
from jax import lax
from jax.experimental import pallas as pl
from jax.experimental.pallas import tpu as pltpu
from jax.experimental.pallas import tpu_sc as plsc

_GROUP = 128
_NBUF = 4


@functools.lru_cache(maxsize=None)
def _make_gather(V, D, B):
  mesh = plsc.VectorSubcoreMesh(core_axis_name="c", subcore_axis_name="s")
  NC, NS = mesh.num_cores, mesh.num_subcores
  NW = NC * NS
  assert B % (NW * _GROUP) == 0
  nidx = B // NW
  ng = nidx // _GROUP
  assert D % 16 == 0
  scale = math.sqrt(D)

  @functools.partial(
      pl.kernel,
      out_type=jax.ShapeDtypeStruct((B, D), jnp.float32),
      mesh=mesh,
      scratch_types=[
          pltpu.VMEM((ng, _GROUP), jnp.int32),
          pltpu.VMEM((_NBUF, _GROUP, D), jnp.float32),
          pltpu.VMEM((_NBUF, _GROUP, D), jnp.float32),
          [pltpu.SemaphoreType.DMA] * _NBUF,
          [pltpu.SemaphoreType.DMA] * _NBUF,
      ],
      compiler_params=pltpu.CompilerParams(use_tc_tiling_on_sc=False),
  )
  def gather_kernel(idx_hbm, table_hbm, out_hbm, idx_v, rin, rout, gsems, osems):
    wid = lax.axis_index("s") * NC + lax.axis_index("c")
    base = wid * nidx

    pltpu.sync_copy(idx_hbm.at[wid], idx_v)

    def fire_gather(g, b):
      pltpu.make_async_copy(
          table_hbm.at[idx_v.at[g]], rin.at[b], gsems[b]).start()

    def wait_gather(g, b):
      pltpu.make_async_copy(
          table_hbm.at[idx_v.at[g]], rin.at[b], gsems[b]).wait()

    def fire_write(g, b):
      pltpu.make_async_copy(
          rout.at[b], out_hbm.at[pl.ds(base + g * _GROUP, _GROUP)],
          osems[b]).start()

    def wait_write(b):
      pltpu.make_async_copy(
          rout.at[b], out_hbm.at[pl.ds(base, _GROUP)], osems[b]).wait()

    for b in range(_NBUF):
      fire_gather(b, b)

    @pl.loop(0, ng, step=_NBUF)
    def _outer(i0):
      for b in range(_NBUF):
        g = i0 + b
        wait_gather(g, b)

        @pl.when(i0 > 0)
        def _():
          wait_write(b)

        @plsc.parallel_loop(0, _GROUP, 1, unroll=8)
        def _scale(r):
          for j in range(D // 16):
            sl = pl.ds(j * 16, 16)
            rout.at[b][r, sl] = rin.at[b][r, sl] * scale

        @pl.when(g + _NBUF < ng)
        def _():
          fire_gather(g + _NBUF, b)

        fire_write(g, b)

    for b in range(_NBUF):
      wait_write(b)

  return gather_kernel, NW, ng


def kernel(tokens, table):
  n, s = tokens.shape
  V, D = table.shape
  B = n * s
  gather_fn, NW, ng = _make_gather(V, D, B)
  idx = tokens.reshape(NW, ng, _GROUP).astype(jnp.int32)
  out = gather_fn(idx, table)
  return out.reshape(n, s, D)


# speedup vs baseline: 1.0011x; 1.0011x over previous
import functools
import math

import jax
import jax.numpy as jnp
from jax import lax
from jax.experimental import pallas as pl
from jax.experimental.pallas import tpu as pltpu
from jax.experimental.pallas import tpu_sc as plsc

_CH = 256
_NBUF = 2


@functools.lru_cache(maxsize=None)
def _make_gather(V, D, N, S):
  mesh = plsc.VectorSubcoreMesh(core_axis_name="c", subcore_axis_name="s")
  NC, NS = mesh.num_cores, mesh.num_subcores
  NW = NC * NS
  assert D % 8 == 0 and N % (NW * _CH) == 0 and _CH % 128 == 0
  JB = D // 8
  NBLK = N // 128
  CPW = N // (NW * _CH)
  nvisit = S * CPW
  assert nvisit % _NBUF == 0
  nb_per_ch = _CH // 128
  scale = math.sqrt(D)

  @functools.partial(
      pl.kernel,
      out_type=jax.ShapeDtypeStruct((S, JB, NBLK, 8, 128), jnp.float32),
      mesh=mesh,
      scratch_types=[
          pltpu.VMEM((S, CPW * _CH), jnp.int32),
          pltpu.VMEM((_NBUF, _CH, D), jnp.float32),
          pltpu.VMEM((_NBUF, JB, nb_per_ch, 8, 128), jnp.float32),
          [pltpu.SemaphoreType.DMA] * _NBUF,
          [pltpu.SemaphoreType.DMA] * _NBUF,
      ],
      compiler_params=pltpu.CompilerParams(
          use_tc_tiling_on_sc=False, needs_layout_passes=False),
  )
  def gather_kernel(tokt_hbm, table_hbm, out_hbm, idx_v, rin, rout, gsems, osems):
    wid = lax.axis_index("s") * NC + lax.axis_index("c")
    nbase = wid * (CPW * _CH)

    pltpu.sync_copy(tokt_hbm.at[:, pl.ds(nbase, CPW * _CH)], idx_v)

    def visit_sc(t):
      s = t // CPW
      cc = t - s * CPW
      return s, cc

    def fire_gather(t, b):
      s, cc = visit_sc(t)
      for j in range(_CH // 128):
        pltpu.make_async_copy(
            table_hbm.at[idx_v.at[s, pl.ds(cc * _CH + j * 128, 128)]],
            rin.at[b, pl.ds(j * 128, 128), :],
            gsems[b]).start()

    def wait_gather(t, b):
      s, cc = visit_sc(t)
      for j in range(_CH // 128):
        pltpu.make_async_copy(
            table_hbm.at[idx_v.at[s, pl.ds(cc * _CH + j * 128, 128)]],
            rin.at[b, pl.ds(j * 128, 128), :],
            gsems[b]).wait()

    def fire_write(t, b):
      s, cc = visit_sc(t)
      nb0 = wid * (CPW * nb_per_ch) + cc * nb_per_ch
      pltpu.make_async_copy(
          rout.at[b], out_hbm.at[s, :, pl.ds(nb0, nb_per_ch)], osems[b]).start()

    def wait_write(t, b):
      s, cc = visit_sc(t)
      nb0 = wid * (CPW * nb_per_ch) + cc * nb_per_ch
      pltpu.make_async_copy(
          rout.at[b], out_hbm.at[s, :, pl.ds(nb0, nb_per_ch)], osems[b]).wait()

    for b in range(_NBUF):
      fire_gather(b, b)

    iota16 = lax.iota(jnp.int32, 16)

    @pl.loop(0, nvisit, step=_NBUF)
    def _outer(t0):
      for b in range(_NBUF):
        t = t0 + b
        wait_gather(t, b)

        @pl.when(t0 > 0)
        def _():
          wait_write(t - _NBUF, b)

        rin_b = rin.at[b]
        rout_b = rout.at[b]

        @plsc.parallel_loop(0, D, 1, unroll=2)
        def _tr(j):
          jb = j // 8
          jj = j - jb * 8
          cols = jnp.full((16,), j, jnp.int32)
          for nbl in range(nb_per_ch):
            for v in range(8):
              rows = iota16 + (nbl * 128 + v * 16)
              vals = plsc.load_gather(rin_b, [rows, cols])
              rout_b[jb, nbl, jj, pl.ds(v * 16, 16)] = vals * scale

        @pl.when(t + _NBUF < nvisit)
        def _():
          fire_gather(t + _NBUF, b)

        fire_write(t, b)

    for b in range(_NBUF):
      wait_write(nvisit - _NBUF + b, b)

  return gather_kernel


def kernel(tokens, table):
  n, s = tokens.shape
  V, D = table.shape
  gather_fn = _make_gather(V, D, n, s)
  tokt = tokens.T.astype(jnp.int32)
  out5 = gather_fn(tokt, table)
  return out5.transpose(2, 4, 0, 1, 3).reshape(n, s, D)
